# baseline (speedup 1.0000x reference)
import jax
import jax.numpy as jnp
from jax.experimental import pallas as pl

B = 4096
D_IN = 784
D_IN_P = 896
N_CLS = 10
H = 1024
EMBED_DIM = 256
LATENT_DIM = 64
LATENT_P = 128
K_CODES = 1024

TILE = 1024


def _body(x_ref, c_ref, we1a_ref, we1b_ref, be1_ref, we2_ref, be2_ref,
          wfc_ref, bfc_ref, cb_ref, wd1_ref, bd1_ref, wd2_ref, bd2_ref,
          recon_ref, ze_ref, zq_ref, probs_ref):
    f32 = jnp.float32
    bf16 = jnp.bfloat16

    def mm(a, b):
        return jnp.dot(a, b, preferred_element_type=f32)

    x = x_ref[...]
    cls = c_ref[...]
    oh = (jax.lax.broadcasted_iota(jnp.int32, (TILE, N_CLS), 1)
          == cls[:, None]).astype(bf16)

    h = mm(x, we1a_ref[...]) + mm(oh, we1b_ref[...])
    h = jnp.maximum(h + be1_ref[...][None, :], 0.0)

    enc = mm(h.astype(bf16), we2_ref[...]) + be2_ref[...][None, :]
    z_e = mm(enc.astype(bf16), wfc_ref[...]) + bfc_ref[...][None, :]
    ze_ref[...] = z_e[:, :LATENT_DIM]

    cb = cb_ref[...]
    cb_sq = jnp.sum(cb * cb, axis=1)[None, :]
    ze_sq = jnp.sum(z_e * z_e, axis=1, keepdims=True)
    cross = jax.lax.dot_general(
        z_e.astype(bf16), cb.astype(bf16),
        (((1,), (1,)), ((), ())), preferred_element_type=f32)
    s = 2.0 * cross - ze_sq - cb_sq
    m = jnp.max(s, axis=1, keepdims=True)
    e = jnp.exp(s - m)
    probs = e / jnp.sum(e, axis=1, keepdims=True)
    probs_ref[...] = probs

    z_q = mm(probs.astype(bf16), cb.astype(bf16))
    zq64 = z_q[:, :LATENT_DIM]
    zq_ref[...] = zq64

    dh = mm(zq64.astype(bf16), wd1_ref[:LATENT_DIM, :])
    dh = dh + mm(oh, wd1_ref[LATENT_DIM:, :])
    dh = jnp.maximum(dh + bd1_ref[...][None, :], 0.0)

    recon = mm(dh.astype(bf16), wd2_ref[...]) + bd2_ref[...][None, :]
    recon_ref[...] = jax.nn.sigmoid(recon)


def kernel(x, c, W_enc1, b_enc1, W_enc2, b_enc2, W_fc, b_fc, codebook,
           W_dec1, b_dec1, W_dec2, b_dec2):
    f32 = jnp.float32
    bf16 = jnp.bfloat16
    pad_c = lambda a, w: jnp.pad(a, ((0, 0), (0, w - a.shape[1])))

    x_bf = pad_c(x, D_IN_P).astype(bf16)
    c32 = c.astype(jnp.int32)
    w1a = jnp.pad(W_enc1[:D_IN], ((0, D_IN_P - D_IN), (0, 0))).astype(bf16)
    w1b = W_enc1[D_IN:].astype(bf16)
    w2 = W_enc2.astype(bf16)
    wfc = pad_c(W_fc, LATENT_P).astype(bf16)
    bfc = jnp.pad(b_fc, (0, LATENT_P - LATENT_DIM))
    cb_p = pad_c(codebook, LATENT_P)
    wd1 = W_dec1.astype(bf16)
    wd2 = pad_c(W_dec2, D_IN_P).astype(bf16)
    bd2 = jnp.pad(b_dec2, (0, D_IN_P - D_IN))

    grid = (B // TILE,)

    def tile2(i):
        return (i, 0)

    def const2(i):
        return (0, 0)

    def const1(i):
        return (0,)

    full = lambda arr: pl.BlockSpec(arr.shape, const2)
    vec = lambda n: pl.BlockSpec((n,), const1)

    out_shapes = (
        jax.ShapeDtypeStruct((B, D_IN_P), f32),
        jax.ShapeDtypeStruct((B, LATENT_DIM), f32),
        jax.ShapeDtypeStruct((B, LATENT_DIM), f32),
        jax.ShapeDtypeStruct((B, K_CODES), f32),
    )
    in_specs = [
        pl.BlockSpec((TILE, D_IN_P), tile2),
        pl.BlockSpec((TILE,), lambda i: (i,)),
        full(w1a), full(w1b), vec(H),
        full(w2), vec(EMBED_DIM),
        full(wfc), vec(LATENT_P),
        full(cb_p),
        full(wd1), vec(H),
        full(wd2), vec(D_IN_P),
    ]
    out_specs = (
        pl.BlockSpec((TILE, D_IN_P), tile2),
        pl.BlockSpec((TILE, LATENT_DIM), tile2),
        pl.BlockSpec((TILE, LATENT_DIM), tile2),
        pl.BlockSpec((TILE, K_CODES), tile2),
    )

    recon_p, z_e, z_q, probs = pl.pallas_call(
        _body,
        grid=grid,
        in_specs=in_specs,
        out_specs=out_specs,
        out_shape=out_shapes,
    )(x_bf, c32, w1a, w1b, b_enc1, w2, b_enc2, wfc, bfc, cb_p,
      wd1, b_dec1, wd2, bd2)
    return (recon_p[:, :D_IN], z_e, z_q, probs)
